# baseline (speedup 1.0000x reference)
import jax
import jax.numpy as jnp
from jax import lax
from jax.experimental import pallas as pl
from jax.experimental.pallas import tpu as pltpu
from jax.experimental.pallas import tpu_sc as plsc

NUM_CORES = 2
NUM_SUBCORES = 16
NW = NUM_CORES * NUM_SUBCORES
LANES = 16
BATCH = 16384
EMBED = 32
TABLE_N = 1_000_000
B_PER_W = BATCH // NW
ICHUNK = 128
N_ICHUNK = B_PER_W // ICHUNK
CSZ = 8
N_GRP = B_PER_W // LANES


def _bpr_body(uids_hbm, iids_hbm, ut_hbm, it_hbm, ub_hbm, ib_hbm, out_hbm,
              uidx, iidx, ubuf, ibuf, ubv, ibv, outv, sem_g, sem_g2, sem_b):
    c = lax.axis_index("c")
    s = lax.axis_index("s")
    wid = s * NUM_CORES + c
    base = wid * B_PER_W

    pltpu.sync_copy(uids_hbm.at[pl.ds(base, B_PER_W)], uidx)
    pltpu.sync_copy(iids_hbm.at[pl.ds(base, B_PER_W)], iidx)

    bias_copies = []
    for j in range(N_ICHUNK):
        isl = pl.ds(j * ICHUNK, ICHUNK)
        bias_copies.append(
            pltpu.async_copy(ub_hbm.at[uidx.at[isl]], ubv.at[isl], sem_b))
        bias_copies.append(
            pltpu.async_copy(ib_hbm.at[iidx.at[isl]], ibv.at[isl], sem_b))
    for cp in bias_copies:
        cp.wait()

    d16 = lax.iota(jnp.int32, LANES)
    lo_s = lax.shift_right_logical(d16, 3)
    lo_d8 = d16 & 7
    hi16 = d16 + LANES
    hi_s = lax.shift_right_logical(hi16, 3)
    hi_d8 = hi16 & 7
    lane16 = d16

    HALF = 4
    sems = (sem_g, sem_g2)

    def fire(uvec, ivec, sub):
        half = sub % 2
        sem = sems[half]
        for i in range(HALF):
            k = sub * HALF + i
            slot = half * HALF + i
            uts = pl.multiple_of(
                lax.shift_left(lax.shift_right_logical(uvec[k], 7), 7), 128)
            its = pl.multiple_of(
                lax.shift_left(lax.shift_right_logical(ivec[k], 7), 7), 128)
            pltpu.async_copy(
                ut_hbm.at[:, :, pl.ds(uts, 128)], ubuf.at[slot], sem)
            pltpu.async_copy(
                it_hbm.at[:, :, pl.ds(its, 128)], ibuf.at[slot], sem)

    def group_body(g, carry):
        goff = pl.multiple_of(g * LANES, LANES)
        uvec = uidx[pl.ds(goff, LANES)]
        ivec = iidx[pl.ds(goff, LANES)]
        ulanes = uvec & 127
        ilanes = ivec & 127
        dvec = jnp.zeros((LANES,), jnp.float32)

        fire(uvec, ivec, 0)
        for sub in range(LANES // HALF):
            half = sub % 2
            if sub + 1 < LANES // HALF:
                fire(uvec, ivec, sub + 1)
            for _ in range(HALF):
                pltpu.make_async_copy(
                    ut_hbm.at[:, :, pl.ds(0, 128)], ubuf.at[0],
                    sems[half]).wait()
                pltpu.make_async_copy(
                    it_hbm.at[:, :, pl.ds(0, 128)], ibuf.at[0],
                    sems[half]).wait()
            for i in range(HALF):
                k = sub * HALF + i
                slot = jnp.full((LANES,), half * HALF + i, jnp.int32)
                ul = jnp.full((LANES,), ulanes[k], jnp.int32)
                il = jnp.full((LANES,), ilanes[k], jnp.int32)
                u_lo = plsc.load_gather(ubuf, [slot, lo_s, lo_d8, ul])
                u_hi = plsc.load_gather(ubuf, [slot, hi_s, hi_d8, ul])
                v_lo = plsc.load_gather(ibuf, [slot, lo_s, lo_d8, il])
                v_hi = plsc.load_gather(ibuf, [slot, hi_s, hi_d8, il])
                p = u_lo * v_lo + u_hi * v_hi
                dot = jnp.sum(p)
                dvec = jnp.where(lane16 == k, jnp.full((LANES,), dot), dvec)

        outv[pl.ds(goff, LANES)] = (
            dvec + ubv[pl.ds(goff, LANES)] + ibv[pl.ds(goff, LANES)])
        return carry

    lax.fori_loop(0, N_GRP, group_body, 0)

    pltpu.sync_copy(outv, out_hbm.at[pl.ds(base, B_PER_W)])


def kernel(user_ids, item_ids, user_embedding, item_embedding, user_bias, item_bias):
    ut = user_embedding.T.reshape(EMBED // 8, 8, TABLE_N)
    it = item_embedding.T.reshape(EMBED // 8, 8, TABLE_N)
    ubias = user_bias.reshape(-1)
    ibias = item_bias.reshape(-1)

    mesh = plsc.VectorSubcoreMesh(
        core_axis_name="c", subcore_axis_name="s",
        num_cores=NUM_CORES, num_subcores=NUM_SUBCORES,
    )
    run = pl.kernel(
        _bpr_body,
        out_type=jax.ShapeDtypeStruct((BATCH,), jnp.float32),
        mesh=mesh,
        scratch_types=[
            pltpu.VMEM((B_PER_W,), jnp.int32),
            pltpu.VMEM((B_PER_W,), jnp.int32),
            pltpu.VMEM((CSZ, EMBED // 8, 8, 128), jnp.float32),
            pltpu.VMEM((CSZ, EMBED // 8, 8, 128), jnp.float32),
            pltpu.VMEM((B_PER_W,), jnp.float32),
            pltpu.VMEM((B_PER_W,), jnp.float32),
            pltpu.VMEM((B_PER_W,), jnp.float32),
            pltpu.SemaphoreType.DMA,
            pltpu.SemaphoreType.DMA,
            pltpu.SemaphoreType.DMA,
        ],
        compiler_params=pltpu.CompilerParams(needs_layout_passes=False),
    )
    return run(user_ids, item_ids, ut, it, ubias, ibias)


# speedup vs baseline: 1.0231x; 1.0231x over previous
import jax
import jax.numpy as jnp
from jax import lax
from jax.experimental import pallas as pl
from jax.experimental.pallas import tpu as pltpu
from jax.experimental.pallas import tpu_sc as plsc

NUM_CORES = 2
NUM_SUBCORES = 16
NW = NUM_CORES * NUM_SUBCORES
LANES = 16
BATCH = 16384
EMBED = 32
TABLE_N = 1_000_000
B_PER_W = BATCH // NW
ICHUNK = 128
N_ICHUNK = B_PER_W // ICHUNK
CSZ = 8
N_GRP = B_PER_W // LANES


def _bpr_body(uids_hbm, iids_hbm, ut_hbm, it_hbm, ub_hbm, ib_hbm, out_hbm,
              uidx, iidx, ubuf, ibuf, ubv, ibv, outv, sem_g, sem_b):
    c = lax.axis_index("c")
    s = lax.axis_index("s")
    wid = s * NUM_CORES + c
    base = wid * B_PER_W

    pltpu.sync_copy(uids_hbm.at[pl.ds(base, B_PER_W)], uidx)
    pltpu.sync_copy(iids_hbm.at[pl.ds(base, B_PER_W)], iidx)

    bias_copies = []
    for j in range(N_ICHUNK):
        isl = pl.ds(j * ICHUNK, ICHUNK)
        bias_copies.append(
            pltpu.async_copy(ub_hbm.at[uidx.at[isl]], ubv.at[isl], sem_b))
        bias_copies.append(
            pltpu.async_copy(ib_hbm.at[iidx.at[isl]], ibv.at[isl], sem_b))
    for cp in bias_copies:
        cp.wait()

    d16 = lax.iota(jnp.int32, LANES)
    lo_s = lax.shift_right_logical(d16, 3)
    lo_d8 = d16 & 7
    hi16 = d16 + LANES
    hi_s = lax.shift_right_logical(hi16, 3)
    hi_d8 = hi16 & 7
    lane16 = d16

    def group_body(g, carry):
        goff = pl.multiple_of(g * LANES, LANES)
        uvec = uidx[pl.ds(goff, LANES)]
        ivec = iidx[pl.ds(goff, LANES)]
        ulanes = uvec & 127
        ilanes = ivec & 127
        dvec = jnp.zeros((LANES,), jnp.float32)

        for sub in range(LANES // CSZ):
            for i in range(CSZ):
                k = sub * CSZ + i
                uts = pl.multiple_of(
                    lax.shift_left(lax.shift_right_logical(uvec[k], 7), 7),
                    128)
                its = pl.multiple_of(
                    lax.shift_left(lax.shift_right_logical(ivec[k], 7), 7),
                    128)
                pltpu.async_copy(
                    ut_hbm.at[:, :, pl.ds(uts, 128)], ubuf.at[i], sem_g)
                pltpu.async_copy(
                    it_hbm.at[:, :, pl.ds(its, 128)], ibuf.at[i], sem_g)
            for i in range(CSZ):
                pltpu.make_async_copy(
                    ut_hbm.at[:, :, pl.ds(0, 128)], ubuf.at[0], sem_g).wait()
                pltpu.make_async_copy(
                    it_hbm.at[:, :, pl.ds(0, 128)], ibuf.at[0], sem_g).wait()
            for i in range(CSZ):
                k = sub * CSZ + i
                slot = jnp.full((LANES,), i, jnp.int32)
                ul = jnp.full((LANES,), ulanes[k], jnp.int32)
                il = jnp.full((LANES,), ilanes[k], jnp.int32)
                u_lo = plsc.load_gather(ubuf, [slot, lo_s, lo_d8, ul])
                u_hi = plsc.load_gather(ubuf, [slot, hi_s, hi_d8, ul])
                v_lo = plsc.load_gather(ibuf, [slot, lo_s, lo_d8, il])
                v_hi = plsc.load_gather(ibuf, [slot, hi_s, hi_d8, il])
                p = u_lo * v_lo + u_hi * v_hi
                dot = jnp.sum(p)
                dvec = jnp.where(lane16 == k, jnp.full((LANES,), dot), dvec)

        outv[pl.ds(goff, LANES)] = (
            dvec + ubv[pl.ds(goff, LANES)] + ibv[pl.ds(goff, LANES)])
        return carry

    lax.fori_loop(0, N_GRP, group_body, 0)

    pltpu.sync_copy(outv, out_hbm.at[pl.ds(base, B_PER_W)])


def kernel(user_ids, item_ids, user_embedding, item_embedding, user_bias, item_bias):
    ut = user_embedding.T.reshape(EMBED // 8, 8, TABLE_N)
    it = item_embedding.T.reshape(EMBED // 8, 8, TABLE_N)
    ubias = user_bias.reshape(-1)
    ibias = item_bias.reshape(-1)

    mesh = plsc.VectorSubcoreMesh(
        core_axis_name="c", subcore_axis_name="s",
        num_cores=NUM_CORES, num_subcores=NUM_SUBCORES,
    )
    run = pl.kernel(
        _bpr_body,
        out_type=jax.ShapeDtypeStruct((BATCH,), jnp.float32),
        mesh=mesh,
        scratch_types=[
            pltpu.VMEM((B_PER_W,), jnp.int32),
            pltpu.VMEM((B_PER_W,), jnp.int32),
            pltpu.VMEM((CSZ, EMBED // 8, 8, 128), jnp.float32),
            pltpu.VMEM((CSZ, EMBED // 8, 8, 128), jnp.float32),
            pltpu.VMEM((B_PER_W,), jnp.float32),
            pltpu.VMEM((B_PER_W,), jnp.float32),
            pltpu.VMEM((B_PER_W,), jnp.float32),
            pltpu.SemaphoreType.DMA,
            pltpu.SemaphoreType.DMA,
        ],
        compiler_params=pltpu.CompilerParams(needs_layout_passes=False),
    )
    return run(user_ids, item_ids, ut, it, ubias, ibias)
